# initial kernel scaffold (unmeasured)
import functools

import jax
import jax.numpy as jnp
from jax import lax
from jax.experimental import pallas as pl
from jax.experimental.pallas import tpu as pltpu

N_DEV = 16
SQ = 2048
D_MODEL = 1024
H_LOC = 8
DH = 128
WIN = 128
QB = 256
KB = 512
CHUNK = SQ // N_DEV
SCALE = 0.08838834764831843


def kernel(x, Wq, K_ext, V_ext, Wo):
    pos = lax.axis_index("i")
    x2 = x[0]
    K = K_ext[0]
    V = V_ext[0]
    Wq_sl = lax.dynamic_slice(Wq, (0, pos * D_MODEL), (D_MODEL, D_MODEL))
    Wo_sl = lax.dynamic_slice(Wo, (pos * D_MODEL, 0), (D_MODEL, D_MODEL))

    def body(x_ref, wq_ref, k_ref, v_ref, wo_ref, out_ref,
             q_ref, ctx_ref, acc_ref, rs_ref, ag_ref,
             send_sem, rs_sems, ag_sems):
        my = lax.axis_index("i")
        left = jnp.mod(my - 1, N_DEV)
        right = jnp.mod(my + 1, N_DEV)

        barrier_sem = pltpu.get_barrier_semaphore()
        for nbr in (left, right):
            pl.semaphore_signal(
                barrier_sem, inc=1,
                device_id=(nbr,), device_id_type=pl.DeviceIdType.MESH,
            )
        pl.semaphore_wait(barrier_sem, 2)

        q_ref[...] = jnp.dot(
            x_ref[...], wq_ref[...], preferred_element_type=jnp.float32
        )
        for qb in range(SQ // QB):
            r0 = qb * QB
            ks = min(max(r0 - WIN, 0), SQ - KB)
            for h in range(H_LOC):
                q = q_ref[r0:r0 + QB, h * DH:(h + 1) * DH]
                k = k_ref[ks:ks + KB, h, :]
                s = lax.dot_general(
                    q, k, (((1,), (1,)), ((), ())),
                    preferred_element_type=jnp.float32,
                ) * SCALE
                rows = r0 + lax.broadcasted_iota(jnp.int32, (QB, KB), 0)
                cols = ks + lax.broadcasted_iota(jnp.int32, (QB, KB), 1)
                s = jnp.where(jnp.abs(rows - cols) <= WIN, s, -1e9)
                m = jnp.max(s, axis=1, keepdims=True)
                e = jnp.exp(s - m)
                w = e / jnp.sum(e, axis=1, keepdims=True)
                vv = v_ref[ks:ks + KB, h, :]
                ctx_ref[r0:r0 + QB, h * DH:(h + 1) * DH] = jnp.dot(
                    w, vv, preferred_element_type=jnp.float32
                )
        acc_ref[...] = jnp.dot(
            ctx_ref[...], wo_ref[...], preferred_element_type=jnp.float32
        )

        for s in range(N_DEV - 1):
            send_c = jnp.mod(my - s, N_DEV)
            rdma = pltpu.make_async_remote_copy(
                src_ref=acc_ref.at[pl.ds(send_c * CHUNK, CHUNK), :],
                dst_ref=rs_ref.at[s],
                send_sem=send_sem,
                recv_sem=rs_sems.at[s],
                device_id=(right,),
                device_id_type=pl.DeviceIdType.MESH,
            )
            rdma.start()
            rdma.wait()
            recv_c = jnp.mod(my - s - 1, N_DEV)
            acc_ref[pl.ds(recv_c * CHUNK, CHUNK), :] = (
                acc_ref[pl.ds(recv_c * CHUNK, CHUNK), :] + rs_ref[s]
            )

        own = jnp.mod(my + 1, N_DEV)
        out_ref[pl.ds(own * CHUNK, CHUNK), :] = (
            acc_ref[pl.ds(own * CHUNK, CHUNK), :]
        )

        for t in range(N_DEV - 1):
            if t == 0:
                src = acc_ref.at[pl.ds(own * CHUNK, CHUNK), :]
            else:
                src = ag_ref.at[t - 1]
            rdma = pltpu.make_async_remote_copy(
                src_ref=src,
                dst_ref=ag_ref.at[t],
                send_sem=send_sem,
                recv_sem=ag_sems.at[t],
                device_id=(right,),
                device_id_type=pl.DeviceIdType.MESH,
            )
            rdma.start()
            rdma.wait()
            c = jnp.mod(my - t, N_DEV)
            out_ref[pl.ds(c * CHUNK, CHUNK), :] = ag_ref[t]

        @functools.partial(
            pl.run_scoped, second_barrier=pltpu.SemaphoreType.REGULAR
        )
        def _(second_barrier):
            for nbr in (left, right):
                pl.semaphore_signal(
                    second_barrier, inc=1,
                    device_id=(nbr,), device_id_type=pl.DeviceIdType.MESH,
                )
            pl.semaphore_wait(second_barrier, 2)

    out2 = pl.pallas_call(
        body,
        out_shape=jax.ShapeDtypeStruct((SQ, D_MODEL), jnp.float32),
        in_specs=[pl.BlockSpec(memory_space=pltpu.VMEM)] * 5,
        out_specs=pl.BlockSpec(memory_space=pltpu.VMEM),
        scratch_shapes=[
            pltpu.VMEM((SQ, D_MODEL), jnp.float32),
            pltpu.VMEM((SQ, D_MODEL), jnp.float32),
            pltpu.VMEM((SQ, D_MODEL), jnp.float32),
            pltpu.VMEM((N_DEV - 1, CHUNK, D_MODEL), jnp.float32),
            pltpu.VMEM((N_DEV - 1, CHUNK, D_MODEL), jnp.float32),
            pltpu.SemaphoreType.DMA,
            pltpu.SemaphoreType.DMA((N_DEV - 1,)),
            pltpu.SemaphoreType.DMA((N_DEV - 1,)),
        ],
        compiler_params=pltpu.CompilerParams(collective_id=0),
    )(x2, Wq_sl, K, V, Wo_sl)
    return out2[None]


# baseline (device time: 306085 ns/iter reference)
import functools

import jax
import jax.numpy as jnp
from jax import lax
from jax.experimental import pallas as pl
from jax.experimental.pallas import tpu as pltpu

N_DEV = 16
SQ = 2048
D_MODEL = 1024
H_LOC = 8
DH = 128
WIN = 128
QB = 256
KB = 512
CHUNK = SQ // N_DEV
SCALE = 0.08838834764831843


def kernel(x, Wq, K_ext, V_ext, Wo):
    pos = lax.axis_index("i")
    x2 = x[0]
    K = K_ext[0]
    V = V_ext[0]
    Wq_sl = lax.dynamic_slice(Wq, (0, pos * D_MODEL), (D_MODEL, D_MODEL))
    Wo_sl = lax.dynamic_slice(Wo, (pos * D_MODEL, 0), (D_MODEL, D_MODEL))

    def body(x_ref, wq_ref, k_ref, v_ref, wo_ref, out_ref,
             stage_ref, rs_ref, ag_ref,
             send_sem, rs_sems, ag_sems):
        my = lax.axis_index("i")
        left = jnp.mod(my - 1, N_DEV)
        right = jnp.mod(my + 1, N_DEV)

        barrier_sem = pltpu.get_barrier_semaphore()
        for nbr in (left, right):
            pl.semaphore_signal(
                barrier_sem, inc=1,
                device_id=(nbr,), device_id_type=pl.DeviceIdType.MESH,
            )
        pl.semaphore_wait(barrier_sem, 2)

        for qb in range(SQ // QB):
            r0 = qb * QB
            ks = min(max(r0 - WIN, 0), SQ - KB)
            q_blk = jnp.dot(
                x_ref[r0:r0 + QB, :], wq_ref[...],
                preferred_element_type=jnp.float32,
            )
            heads = []
            for h in range(H_LOC):
                q = q_blk[:, h * DH:(h + 1) * DH]
                k = k_ref[ks:ks + KB, h, :]
                s = lax.dot_general(
                    q, k, (((1,), (1,)), ((), ())),
                    preferred_element_type=jnp.float32,
                ) * SCALE
                rows = r0 + lax.broadcasted_iota(jnp.int32, (QB, KB), 0)
                cols = ks + lax.broadcasted_iota(jnp.int32, (QB, KB), 1)
                s = jnp.where(jnp.abs(rows - cols) <= WIN, s, -1e9)
                m = jnp.max(s, axis=1, keepdims=True)
                e = jnp.exp(s - m)
                w = e / jnp.sum(e, axis=1, keepdims=True)
                vv = v_ref[ks:ks + KB, h, :]
                heads.append(
                    jnp.dot(w, vv, preferred_element_type=jnp.float32)
                )
            ctx_blk = jnp.concatenate(heads, axis=1)
            out_ref[r0:r0 + QB, :] = jnp.dot(
                ctx_blk, wo_ref[...], preferred_element_type=jnp.float32
            )

        for s in range(N_DEV - 1):
            send_c = jnp.mod(my - s, N_DEV)
            stage_ref[...] = out_ref[pl.ds(send_c * CHUNK, CHUNK), :]
            rdma = pltpu.make_async_remote_copy(
                src_ref=stage_ref,
                dst_ref=rs_ref.at[s],
                send_sem=send_sem,
                recv_sem=rs_sems.at[s],
                device_id=(right,),
                device_id_type=pl.DeviceIdType.MESH,
            )
            rdma.start()
            rdma.wait()
            recv_c = jnp.mod(my - s - 1, N_DEV)
            out_ref[pl.ds(recv_c * CHUNK, CHUNK), :] = (
                out_ref[pl.ds(recv_c * CHUNK, CHUNK), :] + rs_ref[s]
            )

        own = jnp.mod(my + 1, N_DEV)

        for t in range(N_DEV - 1):
            if t == 0:
                stage_ref[...] = out_ref[pl.ds(own * CHUNK, CHUNK), :]
                src = stage_ref
            else:
                src = ag_ref.at[t - 1]
            rdma = pltpu.make_async_remote_copy(
                src_ref=src,
                dst_ref=ag_ref.at[t],
                send_sem=send_sem,
                recv_sem=ag_sems.at[t],
                device_id=(right,),
                device_id_type=pl.DeviceIdType.MESH,
            )
            rdma.start()
            rdma.wait()
            c_recv = jnp.mod(my - t, N_DEV)
            out_ref[pl.ds(c_recv * CHUNK, CHUNK), :] = ag_ref[t]

        @functools.partial(
            pl.run_scoped, second_barrier=pltpu.SemaphoreType.REGULAR
        )
        def _(second_barrier):
            for nbr in (left, right):
                pl.semaphore_signal(
                    second_barrier, inc=1,
                    device_id=(nbr,), device_id_type=pl.DeviceIdType.MESH,
                )
            pl.semaphore_wait(second_barrier, 2)

    out2 = pl.pallas_call(
        body,
        out_shape=jax.ShapeDtypeStruct((SQ, D_MODEL), jnp.float32),
        in_specs=[pl.BlockSpec(memory_space=pltpu.VMEM)] * 5,
        out_specs=pl.BlockSpec(memory_space=pltpu.VMEM),
        scratch_shapes=[
            pltpu.VMEM((CHUNK, D_MODEL), jnp.float32),
            pltpu.VMEM((N_DEV - 1, CHUNK, D_MODEL), jnp.float32),
            pltpu.VMEM((N_DEV - 1, CHUNK, D_MODEL), jnp.float32),
            pltpu.SemaphoreType.DMA,
            pltpu.SemaphoreType.DMA((N_DEV - 1,)),
            pltpu.SemaphoreType.DMA((N_DEV - 1,)),
        ],
        compiler_params=pltpu.CompilerParams(
            collective_id=0, vmem_limit_bytes=100 * 1024 * 1024
        ),
    )(x2, Wq_sl, K, V, Wo_sl)
    return out2[None]


# device time: 261859 ns/iter; 1.1689x vs baseline; 1.1689x over previous
import functools

import jax
import jax.numpy as jnp
from jax import lax
from jax.experimental import pallas as pl
from jax.experimental.pallas import tpu as pltpu

N_DEV = 16
SQ = 2048
D_MODEL = 1024
H_LOC = 8
DH = 128
WIN = 128
QB = 256
KB = 512
HALF = SQ // 2
CH = HALF // N_DEV
SCALE = 0.08838834764831843


def kernel(x, Wq, K_ext, V_ext, Wo):
    pos = lax.axis_index("i")
    x2 = x[0]
    K = K_ext[0]
    V = V_ext[0]
    Wq_sl = lax.dynamic_slice(Wq, (0, pos * D_MODEL), (D_MODEL, D_MODEL))
    Wo_sl = lax.dynamic_slice(Wo, (pos * D_MODEL, 0), (D_MODEL, D_MODEL))

    def body(x_ref, wq_ref, k_ref, v_ref, wo_ref, out_ref,
             stage_r, stage_l, rs_r, rs_l, ag_r, ag_l,
             send_sem_r, send_sem_l, rs_sems_r, rs_sems_l,
             ag_sems_r, ag_sems_l):
        my = lax.axis_index("i")
        left = jnp.mod(my - 1, N_DEV)
        right = jnp.mod(my + 1, N_DEV)

        barrier_sem = pltpu.get_barrier_semaphore()
        for nbr in (left, right):
            pl.semaphore_signal(
                barrier_sem, inc=1,
                device_id=(nbr,), device_id_type=pl.DeviceIdType.MESH,
            )
        pl.semaphore_wait(barrier_sem, 2)

        for qb in range(SQ // QB):
            r0 = qb * QB
            ks = min(max(r0 - WIN, 0), SQ - KB)
            q_blk = jnp.dot(
                x_ref[r0:r0 + QB, :], wq_ref[...],
                preferred_element_type=jnp.float32,
            )
            heads = []
            for h in range(H_LOC):
                q = q_blk[:, h * DH:(h + 1) * DH]
                k = k_ref[ks:ks + KB, h, :]
                s = lax.dot_general(
                    q, k, (((1,), (1,)), ((), ())),
                    preferred_element_type=jnp.float32,
                ) * SCALE
                rows = r0 + lax.broadcasted_iota(jnp.int32, (QB, KB), 0)
                cols = ks + lax.broadcasted_iota(jnp.int32, (QB, KB), 1)
                s = jnp.where(jnp.abs(rows - cols) <= WIN, s, -1e9)
                m = jnp.max(s, axis=1, keepdims=True)
                e = jnp.exp(s - m)
                w = e / jnp.sum(e, axis=1, keepdims=True)
                vv = v_ref[ks:ks + KB, h, :]
                heads.append(
                    jnp.dot(w, vv, preferred_element_type=jnp.float32)
                )
            ctx_blk = jnp.concatenate(heads, axis=1)
            out_ref[r0:r0 + QB, :] = jnp.dot(
                ctx_blk, wo_ref[...], preferred_element_type=jnp.float32
            )

        def mk(src, dst, ssem, rsem, nbr):
            return pltpu.make_async_remote_copy(
                src_ref=src, dst_ref=dst, send_sem=ssem, recv_sem=rsem,
                device_id=(nbr,), device_id_type=pl.DeviceIdType.MESH,
            )

        for s in range(N_DEV - 1):
            send_cr = jnp.mod(my - s, N_DEV)
            send_cl = jnp.mod(my + s, N_DEV)
            stage_r[...] = out_ref[pl.ds(send_cr * CH, CH), :]
            stage_l[...] = out_ref[pl.ds(HALF + send_cl * CH, CH), :]
            rdma_r = mk(stage_r, rs_r.at[s], send_sem_r, rs_sems_r.at[s],
                        right)
            rdma_l = mk(stage_l, rs_l.at[s], send_sem_l, rs_sems_l.at[s],
                        left)
            rdma_r.start()
            rdma_l.start()
            rdma_r.wait()
            rdma_l.wait()
            recv_cr = jnp.mod(my - s - 1, N_DEV)
            recv_cl = jnp.mod(my + s + 1, N_DEV)
            out_ref[pl.ds(recv_cr * CH, CH), :] = (
                out_ref[pl.ds(recv_cr * CH, CH), :] + rs_r[s]
            )
            out_ref[pl.ds(HALF + recv_cl * CH, CH), :] = (
                out_ref[pl.ds(HALF + recv_cl * CH, CH), :] + rs_l[s]
            )

        own_r = jnp.mod(my + 1, N_DEV)
        own_l = jnp.mod(my - 1, N_DEV)

        for t in range(N_DEV - 1):
            if t == 0:
                stage_r[...] = out_ref[pl.ds(own_r * CH, CH), :]
                stage_l[...] = out_ref[pl.ds(HALF + own_l * CH, CH), :]
                src_r, src_l = stage_r, stage_l
            else:
                src_r, src_l = ag_r.at[t - 1], ag_l.at[t - 1]
            rdma_r = mk(src_r, ag_r.at[t], send_sem_r, ag_sems_r.at[t],
                        right)
            rdma_l = mk(src_l, ag_l.at[t], send_sem_l, ag_sems_l.at[t],
                        left)
            rdma_r.start()
            rdma_l.start()
            rdma_r.wait()
            rdma_l.wait()
            c_rr = jnp.mod(my - t, N_DEV)
            c_rl = jnp.mod(my + t, N_DEV)
            out_ref[pl.ds(c_rr * CH, CH), :] = ag_r[t]
            out_ref[pl.ds(HALF + c_rl * CH, CH), :] = ag_l[t]

        @functools.partial(
            pl.run_scoped, second_barrier=pltpu.SemaphoreType.REGULAR
        )
        def _(second_barrier):
            for nbr in (left, right):
                pl.semaphore_signal(
                    second_barrier, inc=1,
                    device_id=(nbr,), device_id_type=pl.DeviceIdType.MESH,
                )
            pl.semaphore_wait(second_barrier, 2)

    out2 = pl.pallas_call(
        body,
        out_shape=jax.ShapeDtypeStruct((SQ, D_MODEL), jnp.float32),
        in_specs=[pl.BlockSpec(memory_space=pltpu.VMEM)] * 5,
        out_specs=pl.BlockSpec(memory_space=pltpu.VMEM),
        scratch_shapes=[
            pltpu.VMEM((CH, D_MODEL), jnp.float32),
            pltpu.VMEM((CH, D_MODEL), jnp.float32),
            pltpu.VMEM((N_DEV - 1, CH, D_MODEL), jnp.float32),
            pltpu.VMEM((N_DEV - 1, CH, D_MODEL), jnp.float32),
            pltpu.VMEM((N_DEV - 1, CH, D_MODEL), jnp.float32),
            pltpu.VMEM((N_DEV - 1, CH, D_MODEL), jnp.float32),
            pltpu.SemaphoreType.DMA,
            pltpu.SemaphoreType.DMA,
            pltpu.SemaphoreType.DMA((N_DEV - 1,)),
            pltpu.SemaphoreType.DMA((N_DEV - 1,)),
            pltpu.SemaphoreType.DMA((N_DEV - 1,)),
            pltpu.SemaphoreType.DMA((N_DEV - 1,)),
        ],
        compiler_params=pltpu.CompilerParams(
            collective_id=0, vmem_limit_bytes=100 * 1024 * 1024
        ),
    )(x2, Wq_sl, K, V, Wo_sl)
    return out2[None]


# device time: 247210 ns/iter; 1.2382x vs baseline; 1.0593x over previous
import functools

import jax
import jax.numpy as jnp
from jax import lax
from jax.experimental import pallas as pl
from jax.experimental.pallas import tpu as pltpu

N_DEV = 16
SQ = 2048
D_MODEL = 1024
H_LOC = 8
DH = 128
WIN = 128
QB = 256
KB = 512
HALF = SQ // 2
CH = HALF // N_DEV
SCALE = 0.08838834764831843


def kernel(x, Wq, K_ext, V_ext, Wo):
    pos = lax.axis_index("i")
    x2 = x[0]
    K = K_ext[0]
    V = V_ext[0]
    Wq_sl = lax.dynamic_slice(Wq, (0, pos * D_MODEL), (D_MODEL, D_MODEL))
    Wo_sl = lax.dynamic_slice(Wo, (pos * D_MODEL, 0), (D_MODEL, D_MODEL))

    def body(x_ref, wq_ref, k_ref, v_ref, wo_ref, out_ref,
             stage_r, stage_l, rs_r, rs_l, ag_r, ag_l,
             send_sem_r, send_sem_l, rs_sems_r, rs_sems_l,
             ag_sems_r, ag_sems_l):
        my = lax.axis_index("i")
        left = jnp.mod(my - 1, N_DEV)
        right = jnp.mod(my + 1, N_DEV)

        barrier_sem = pltpu.get_barrier_semaphore()
        for nbr in (left, right):
            pl.semaphore_signal(
                barrier_sem, inc=1,
                device_id=(nbr,), device_id_type=pl.DeviceIdType.MESH,
            )

        def compute_block(r0):
            ks = jnp.clip(r0 - WIN, 0, SQ - KB)
            q_blk = jnp.dot(
                x_ref[pl.ds(r0, QB), :], wq_ref[...],
                preferred_element_type=jnp.float32,
            )
            heads = []
            rows = r0 + lax.broadcasted_iota(jnp.int32, (QB, KB), 0)
            cols = ks + lax.broadcasted_iota(jnp.int32, (QB, KB), 1)
            band = jnp.abs(rows - cols) <= WIN
            for h in range(H_LOC):
                q = q_blk[:, h * DH:(h + 1) * DH]
                k = k_ref[pl.ds(ks, KB), h, :]
                s = lax.dot_general(
                    q, k, (((1,), (1,)), ((), ())),
                    preferred_element_type=jnp.float32,
                ) * SCALE
                s = jnp.where(band, s, -1e9)
                m = jnp.max(s, axis=1, keepdims=True)
                e = jnp.exp(s - m)
                w = e / jnp.sum(e, axis=1, keepdims=True)
                vv = v_ref[pl.ds(ks, KB), h, :]
                heads.append(
                    jnp.dot(w, vv, preferred_element_type=jnp.float32)
                )
            ctx_blk = jnp.concatenate(heads, axis=1)
            out_ref[pl.ds(r0, QB), :] = jnp.dot(
                ctx_blk, wo_ref[...], preferred_element_type=jnp.float32
            )

        def rblock(j):
            return jnp.mod(my // 4 - j, 4) * QB

        def lblock(j):
            return HALF + jnp.mod(my // 4 + j, 4) * QB

        for j in (0, 1):
            compute_block(rblock(j))
            compute_block(lblock(j))

        def mk(src, dst, ssem, rsem, nbr):
            return pltpu.make_async_remote_copy(
                src_ref=src, dst_ref=dst, send_sem=ssem, recv_sem=rsem,
                device_id=(nbr,), device_id_type=pl.DeviceIdType.MESH,
            )

        pl.semaphore_wait(barrier_sem, 2)

        stage_r[0, :, :] = out_ref[pl.ds(jnp.mod(my, N_DEV) * CH, CH), :]
        stage_l[0, :, :] = out_ref[
            pl.ds(HALF + jnp.mod(my, N_DEV) * CH, CH), :
        ]
        prev_r = mk(stage_r.at[0], rs_r.at[0], send_sem_r,
                    rs_sems_r.at[0], right)
        prev_l = mk(stage_l.at[0], rs_l.at[0], send_sem_l,
                    rs_sems_l.at[0], left)
        prev_r.start()
        prev_l.start()
        for s in range(N_DEV - 1):
            if s == 4:
                compute_block(rblock(2))
                compute_block(lblock(2))
            if s == 8:
                compute_block(rblock(3))
                compute_block(lblock(3))
            prev_r.wait_recv()
            prev_l.wait_recv()
            acc_cr = jnp.mod(my - s - 1, N_DEV)
            acc_cl = jnp.mod(my + s + 1, N_DEV)
            val_r = out_ref[pl.ds(acc_cr * CH, CH), :] + rs_r[s]
            val_l = out_ref[pl.ds(HALF + acc_cl * CH, CH), :] + rs_l[s]
            if s < N_DEV - 2:
                b = (s + 1) % 2
                stage_r[b, :, :] = val_r
                stage_l[b, :, :] = val_l
                prev_r.wait_send()
                prev_l.wait_send()
                nxt_r = mk(stage_r.at[b], rs_r.at[s + 1], send_sem_r,
                           rs_sems_r.at[s + 1], right)
                nxt_l = mk(stage_l.at[b], rs_l.at[s + 1], send_sem_l,
                           rs_sems_l.at[s + 1], left)
                nxt_r.start()
                nxt_l.start()
                prev_r, prev_l = nxt_r, nxt_l
            else:
                out_ref[pl.ds(acc_cr * CH, CH), :] = val_r
                out_ref[pl.ds(HALF + acc_cl * CH, CH), :] = val_l
                stage_r[1, :, :] = val_r
                stage_l[1, :, :] = val_l
                prev_r.wait_send()
                prev_l.wait_send()

        prev_r = mk(stage_r.at[1], ag_r.at[0], send_sem_r,
                    ag_sems_r.at[0], right)
        prev_l = mk(stage_l.at[1], ag_l.at[0], send_sem_l,
                    ag_sems_l.at[0], left)
        prev_r.start()
        prev_l.start()
        for t in range(N_DEV - 1):
            prev_r.wait_recv()
            prev_l.wait_recv()
            if t < N_DEV - 2:
                prev_r.wait_send()
                prev_l.wait_send()
                nxt_r = mk(ag_r.at[t], ag_r.at[t + 1], send_sem_r,
                           ag_sems_r.at[t + 1], right)
                nxt_l = mk(ag_l.at[t], ag_l.at[t + 1], send_sem_l,
                           ag_sems_l.at[t + 1], left)
                nxt_r.start()
                nxt_l.start()
                prev_r, prev_l = nxt_r, nxt_l
            else:
                prev_r.wait_send()
                prev_l.wait_send()
            c_rr = jnp.mod(my - t, N_DEV)
            c_rl = jnp.mod(my + t, N_DEV)
            out_ref[pl.ds(c_rr * CH, CH), :] = ag_r[t]
            out_ref[pl.ds(HALF + c_rl * CH, CH), :] = ag_l[t]

        @functools.partial(
            pl.run_scoped, second_barrier=pltpu.SemaphoreType.REGULAR
        )
        def _(second_barrier):
            for nbr in (left, right):
                pl.semaphore_signal(
                    second_barrier, inc=1,
                    device_id=(nbr,), device_id_type=pl.DeviceIdType.MESH,
                )
            pl.semaphore_wait(second_barrier, 2)

    out2 = pl.pallas_call(
        body,
        out_shape=jax.ShapeDtypeStruct((SQ, D_MODEL), jnp.float32),
        in_specs=[pl.BlockSpec(memory_space=pltpu.VMEM)] * 5,
        out_specs=pl.BlockSpec(memory_space=pltpu.VMEM),
        scratch_shapes=[
            pltpu.VMEM((2, CH, D_MODEL), jnp.float32),
            pltpu.VMEM((2, CH, D_MODEL), jnp.float32),
            pltpu.VMEM((N_DEV - 1, CH, D_MODEL), jnp.float32),
            pltpu.VMEM((N_DEV - 1, CH, D_MODEL), jnp.float32),
            pltpu.VMEM((N_DEV - 1, CH, D_MODEL), jnp.float32),
            pltpu.VMEM((N_DEV - 1, CH, D_MODEL), jnp.float32),
            pltpu.SemaphoreType.DMA,
            pltpu.SemaphoreType.DMA,
            pltpu.SemaphoreType.DMA((N_DEV - 1,)),
            pltpu.SemaphoreType.DMA((N_DEV - 1,)),
            pltpu.SemaphoreType.DMA((N_DEV - 1,)),
            pltpu.SemaphoreType.DMA((N_DEV - 1,)),
        ],
        compiler_params=pltpu.CompilerParams(
            collective_id=0, vmem_limit_bytes=100 * 1024 * 1024
        ),
    )(x2, Wq_sl, K, V, Wo_sl)
    return out2[None]


# device time: 189536 ns/iter; 1.6149x vs baseline; 1.3043x over previous
import functools

import jax
import jax.numpy as jnp
from jax import lax
from jax.experimental import pallas as pl
from jax.experimental.pallas import tpu as pltpu

N_DEV = 16
SQ = 2048
D_MODEL = 1024
H_LOC = 8
DH = 128
WIN = 128
MB = 128
KBM = 384
HALF = SQ // 2
CH = HALF // N_DEV
NMB = HALF // MB
SCALE = 0.08838834764831843


def kernel(x, Wq, K_ext, V_ext, Wo):
    pos = lax.axis_index("i")
    x2 = x[0]
    K = K_ext[0]
    V = V_ext[0]
    Wq_sl = lax.dynamic_slice(Wq, (0, pos * D_MODEL), (D_MODEL, D_MODEL))
    Wo_sl = lax.dynamic_slice(Wo, (pos * D_MODEL, 0), (D_MODEL, D_MODEL))

    def body(x_ref, wq_ref, k_ref, v_ref, wo_ref, out_ref,
             stage_r, stage_l, rs_r, rs_l, ag_r, ag_l,
             send_sem_r, send_sem_l, rs_sems_r, rs_sems_l,
             ag_sems_r, ag_sems_l):
        my = lax.axis_index("i")
        left = jnp.mod(my - 1, N_DEV)
        right = jnp.mod(my + 1, N_DEV)

        barrier_sem = pltpu.get_barrier_semaphore()
        for nbr in (left, right):
            pl.semaphore_signal(
                barrier_sem, inc=1,
                device_id=(nbr,), device_id_type=pl.DeviceIdType.MESH,
            )

        def compute_mini(r0):
            ks = jnp.clip(r0 - WIN, 0, SQ - KBM)
            q_blk = jnp.dot(
                x_ref[pl.ds(r0, MB), :], wq_ref[...],
                preferred_element_type=jnp.float32,
            )
            heads = []
            rows = r0 + lax.broadcasted_iota(jnp.int32, (MB, KBM), 0)
            cols = ks + lax.broadcasted_iota(jnp.int32, (MB, KBM), 1)
            band = jnp.abs(rows - cols) <= WIN
            for h in range(H_LOC):
                q = q_blk[:, h * DH:(h + 1) * DH]
                k = k_ref[pl.ds(ks, KBM), h, :]
                s = lax.dot_general(
                    q, k, (((1,), (1,)), ((), ())),
                    preferred_element_type=jnp.float32,
                ) * SCALE
                s = jnp.where(band, s, -1e9)
                m = jnp.max(s, axis=1, keepdims=True)
                e = jnp.exp(s - m)
                w = e / jnp.sum(e, axis=1, keepdims=True)
                vv = v_ref[pl.ds(ks, KBM), h, :]
                heads.append(
                    jnp.dot(w, vv, preferred_element_type=jnp.float32)
                )
            ctx_blk = jnp.concatenate(heads, axis=1)
            out_ref[pl.ds(r0, MB), :] = jnp.dot(
                ctx_blk, wo_ref[...], preferred_element_type=jnp.float32
            )

        def mb_r(j):
            return jnp.mod(my // 2 - j, NMB) * MB

        def mb_l(j):
            return HALF + jnp.mod(my // 2 + j, NMB) * MB

        for j in (0, 1):
            compute_mini(mb_r(j))
            compute_mini(mb_l(j))

        def mk(src, dst, ssem, rsem, nbr):
            return pltpu.make_async_remote_copy(
                src_ref=src, dst_ref=dst, send_sem=ssem, recv_sem=rsem,
                device_id=(nbr,), device_id_type=pl.DeviceIdType.MESH,
            )

        pl.semaphore_wait(barrier_sem, 2)

        stage_r[0, :, :] = out_ref[pl.ds(my * CH, CH), :].astype(jnp.bfloat16)
        stage_l[0, :, :] = out_ref[pl.ds(HALF + my * CH, CH), :].astype(
            jnp.bfloat16
        )
        prev_r = mk(stage_r.at[0], rs_r.at[0], send_sem_r,
                    rs_sems_r.at[0], right)
        prev_l = mk(stage_l.at[0], rs_l.at[0], send_sem_l,
                    rs_sems_l.at[0], left)
        prev_r.start()
        prev_l.start()
        for s in range(N_DEV - 1):
            if s % 2 == 1 and s <= 11:
                compute_mini(mb_r((s + 3) // 2))
            if s % 2 == 0 and 2 <= s <= 12:
                compute_mini(mb_l(s // 2 + 1))
            prev_r.wait_recv()
            prev_l.wait_recv()
            acc_cr = jnp.mod(my - s - 1, N_DEV)
            acc_cl = jnp.mod(my + s + 1, N_DEV)
            val_r = (out_ref[pl.ds(acc_cr * CH, CH), :]
                     + rs_r[s].astype(jnp.float32))
            val_l = (out_ref[pl.ds(HALF + acc_cl * CH, CH), :]
                     + rs_l[s].astype(jnp.float32))
            if s < N_DEV - 2:
                b = (s + 1) % 2
                stage_r[b, :, :] = val_r.astype(jnp.bfloat16)
                stage_l[b, :, :] = val_l.astype(jnp.bfloat16)
                prev_r.wait_send()
                prev_l.wait_send()
                nxt_r = mk(stage_r.at[b], rs_r.at[s + 1], send_sem_r,
                           rs_sems_r.at[s + 1], right)
                nxt_l = mk(stage_l.at[b], rs_l.at[s + 1], send_sem_l,
                           rs_sems_l.at[s + 1], left)
                nxt_r.start()
                nxt_l.start()
                prev_r, prev_l = nxt_r, nxt_l
            else:
                out_ref[pl.ds(acc_cr * CH, CH), :] = val_r
                out_ref[pl.ds(HALF + acc_cl * CH, CH), :] = val_l
                stage_r[1, :, :] = val_r.astype(jnp.bfloat16)
                stage_l[1, :, :] = val_l.astype(jnp.bfloat16)
                prev_r.wait_send()
                prev_l.wait_send()

        prev_r = mk(stage_r.at[1], ag_r.at[0], send_sem_r,
                    ag_sems_r.at[0], right)
        prev_l = mk(stage_l.at[1], ag_l.at[0], send_sem_l,
                    ag_sems_l.at[0], left)
        prev_r.start()
        prev_l.start()
        for t in range(N_DEV - 1):
            prev_r.wait_recv()
            prev_l.wait_recv()
            prev_r.wait_send()
            prev_l.wait_send()
            if t < N_DEV - 2:
                nxt_r = mk(ag_r.at[t], ag_r.at[t + 1], send_sem_r,
                           ag_sems_r.at[t + 1], right)
                nxt_l = mk(ag_l.at[t], ag_l.at[t + 1], send_sem_l,
                           ag_sems_l.at[t + 1], left)
                nxt_r.start()
                nxt_l.start()
                prev_r, prev_l = nxt_r, nxt_l
            c_rr = jnp.mod(my - t, N_DEV)
            c_rl = jnp.mod(my + t, N_DEV)
            out_ref[pl.ds(c_rr * CH, CH), :] = ag_r[t].astype(jnp.float32)
            out_ref[pl.ds(HALF + c_rl * CH, CH), :] = ag_l[t].astype(
                jnp.float32
            )

        @functools.partial(
            pl.run_scoped, second_barrier=pltpu.SemaphoreType.REGULAR
        )
        def _(second_barrier):
            for nbr in (left, right):
                pl.semaphore_signal(
                    second_barrier, inc=1,
                    device_id=(nbr,), device_id_type=pl.DeviceIdType.MESH,
                )
            pl.semaphore_wait(second_barrier, 2)

    out2 = pl.pallas_call(
        body,
        out_shape=jax.ShapeDtypeStruct((SQ, D_MODEL), jnp.float32),
        in_specs=[pl.BlockSpec(memory_space=pltpu.VMEM)] * 5,
        out_specs=pl.BlockSpec(memory_space=pltpu.VMEM),
        scratch_shapes=[
            pltpu.VMEM((2, CH, D_MODEL), jnp.bfloat16),
            pltpu.VMEM((2, CH, D_MODEL), jnp.bfloat16),
            pltpu.VMEM((N_DEV - 1, CH, D_MODEL), jnp.bfloat16),
            pltpu.VMEM((N_DEV - 1, CH, D_MODEL), jnp.bfloat16),
            pltpu.VMEM((N_DEV - 1, CH, D_MODEL), jnp.bfloat16),
            pltpu.VMEM((N_DEV - 1, CH, D_MODEL), jnp.bfloat16),
            pltpu.SemaphoreType.DMA,
            pltpu.SemaphoreType.DMA,
            pltpu.SemaphoreType.DMA((N_DEV - 1,)),
            pltpu.SemaphoreType.DMA((N_DEV - 1,)),
            pltpu.SemaphoreType.DMA((N_DEV - 1,)),
            pltpu.SemaphoreType.DMA((N_DEV - 1,)),
        ],
        compiler_params=pltpu.CompilerParams(
            collective_id=0, vmem_limit_bytes=100 * 1024 * 1024
        ),
    )(x2, Wq_sl, K, V, Wo_sl)
    return out2[None]


# device time: 176031 ns/iter; 1.7388x vs baseline; 1.0767x over previous
import functools

import jax
import jax.numpy as jnp
from jax import lax
from jax.experimental import pallas as pl
from jax.experimental.pallas import tpu as pltpu

N_DEV = 16
SQ = 2048
D_MODEL = 1024
H_LOC = 8
DH = 128
WIN = 128
MB = 128
KBM = 384
HALF = SQ // 2
CH = HALF // N_DEV
NMB = HALF // MB
SCALE = 0.08838834764831843


def kernel(x, Wq, K_ext, V_ext, Wo):
    pos = lax.axis_index("i")
    x2 = x[0]
    K = K_ext[0]
    V = V_ext[0]
    Wq_sl = lax.dynamic_slice(Wq, (0, pos * D_MODEL), (D_MODEL, D_MODEL))
    Wo_sl = lax.dynamic_slice(Wo, (pos * D_MODEL, 0), (D_MODEL, D_MODEL))

    def body(x_ref, wq_ref, k_ref, v_ref, wo_ref, out_ref,
             stage_r, stage_l, rs_r, rs_l, ag_r, ag_l,
             send_sem_r, send_sem_l, rs_sems_r, rs_sems_l,
             ag_sems_r, ag_sems_l):
        my = lax.axis_index("i")
        left = jnp.mod(my - 1, N_DEV)
        right = jnp.mod(my + 1, N_DEV)

        barrier_sem = pltpu.get_barrier_semaphore()
        for nbr in (left, right):
            pl.semaphore_signal(
                barrier_sem, inc=1,
                device_id=(nbr,), device_id_type=pl.DeviceIdType.MESH,
            )

        def compute_mini(r0):
            ks = jnp.clip(r0 - WIN, 0, SQ - KBM)
            q_blk = jnp.dot(
                x_ref[pl.ds(r0, MB), :], wq_ref[...],
                preferred_element_type=jnp.float32,
            )
            heads = []
            rows = r0 + lax.broadcasted_iota(jnp.int32, (MB, KBM), 0)
            cols = ks + lax.broadcasted_iota(jnp.int32, (MB, KBM), 1)
            band = jnp.abs(rows - cols) <= WIN
            for h in range(H_LOC):
                q = q_blk[:, h * DH:(h + 1) * DH]
                k = k_ref[pl.ds(ks, KBM), h, :]
                s = lax.dot_general(
                    q, k, (((1,), (1,)), ((), ())),
                    preferred_element_type=jnp.float32,
                ) * SCALE
                s = jnp.where(band, s, -1e9)
                m = jnp.max(s, axis=1, keepdims=True)
                e = jnp.exp(s - m)
                w = e / jnp.sum(e, axis=1, keepdims=True)
                vv = v_ref[pl.ds(ks, KBM), h, :]
                heads.append(
                    jnp.dot(w, vv, preferred_element_type=jnp.float32)
                )
            ctx_blk = jnp.concatenate(heads, axis=1)
            out_ref[pl.ds(r0, MB), :] = jnp.dot(
                ctx_blk, wo_ref[...], preferred_element_type=jnp.float32
            )

        def mb_r(j):
            return jnp.mod(my // 2 - j, NMB) * MB

        def mb_l(j):
            return HALF + jnp.mod(my // 2 + j, NMB) * MB

        compute_mini(mb_r(0))
        compute_mini(mb_l(0))

        def mk(src, dst, ssem, rsem, nbr):
            return pltpu.make_async_remote_copy(
                src_ref=src, dst_ref=dst, send_sem=ssem, recv_sem=rsem,
                device_id=(nbr,), device_id_type=pl.DeviceIdType.MESH,
            )

        pl.semaphore_wait(barrier_sem, 2)

        stage_r[0, :, :] = out_ref[pl.ds(my * CH, CH), :].astype(jnp.bfloat16)
        stage_l[0, :, :] = out_ref[pl.ds(HALF + my * CH, CH), :].astype(
            jnp.bfloat16
        )
        prev_r = mk(stage_r.at[0], rs_r.at[0], send_sem_r.at[0],
                    rs_sems_r.at[0], right)
        prev_l = mk(stage_l.at[0], rs_l.at[0], send_sem_l.at[0],
                    rs_sems_l.at[0], left)
        prev_r.start()
        prev_l.start()
        for s in range(N_DEV - 1):
            if s == 0:
                compute_mini(mb_r(1))
                compute_mini(mb_l(1))
            if s % 2 == 1 and s <= 11:
                compute_mini(mb_r((s + 3) // 2))
            if s % 2 == 0 and 2 <= s <= 12:
                compute_mini(mb_l(s // 2 + 1))
            prev_r.wait_recv()
            prev_l.wait_recv()
            acc_cr = jnp.mod(my - s - 1, N_DEV)
            acc_cl = jnp.mod(my + s + 1, N_DEV)
            val_r = (out_ref[pl.ds(acc_cr * CH, CH), :]
                     + rs_r[s].astype(jnp.float32))
            val_l = (out_ref[pl.ds(HALF + acc_cl * CH, CH), :]
                     + rs_l[s].astype(jnp.float32))
            if s < N_DEV - 2:
                b = (s + 1) % 2
                stage_r[b, :, :] = val_r.astype(jnp.bfloat16)
                stage_l[b, :, :] = val_l.astype(jnp.bfloat16)
                prev_r.wait_send()
                prev_l.wait_send()
                nxt_r = mk(stage_r.at[b], rs_r.at[s + 1], send_sem_r.at[0],
                           rs_sems_r.at[s + 1], right)
                nxt_l = mk(stage_l.at[b], rs_l.at[s + 1], send_sem_l.at[0],
                           rs_sems_l.at[s + 1], left)
                nxt_r.start()
                nxt_l.start()
                prev_r, prev_l = nxt_r, nxt_l
            else:
                out_ref[pl.ds(acc_cr * CH, CH), :] = val_r
                out_ref[pl.ds(HALF + acc_cl * CH, CH), :] = val_l
                stage_r[1, :, :] = val_r.astype(jnp.bfloat16)
                stage_l[1, :, :] = val_l.astype(jnp.bfloat16)
                prev_r.wait_send()
                prev_l.wait_send()

        CSPL = D_MODEL // 2
        cols = (slice(0, CSPL), slice(CSPL, D_MODEL))
        prevs = []
        for c in (0, 1):
            pr = mk(stage_r.at[1, :, cols[c]], ag_r.at[0, :, cols[c]],
                    send_sem_r.at[c], ag_sems_r.at[0, c], right)
            pl_ = mk(stage_l.at[1, :, cols[c]], ag_l.at[0, :, cols[c]],
                     send_sem_l.at[c], ag_sems_l.at[0, c], left)
            pr.start()
            pl_.start()
            prevs.append([pr, pl_])
        for t in range(N_DEV - 1):
            for c in (0, 1):
                pr, pl_ = prevs[c]
                pr.wait_recv()
                pl_.wait_recv()
                pr.wait_send()
                pl_.wait_send()
                if t < N_DEV - 2:
                    nr = mk(ag_r.at[t, :, cols[c]],
                            ag_r.at[t + 1, :, cols[c]],
                            send_sem_r.at[c], ag_sems_r.at[t + 1, c],
                            right)
                    nl = mk(ag_l.at[t, :, cols[c]],
                            ag_l.at[t + 1, :, cols[c]],
                            send_sem_l.at[c], ag_sems_l.at[t + 1, c],
                            left)
                    nr.start()
                    nl.start()
                    prevs[c] = [nr, nl]
            c_rr = jnp.mod(my - t, N_DEV)
            c_rl = jnp.mod(my + t, N_DEV)
            out_ref[pl.ds(c_rr * CH, CH), :] = ag_r[t].astype(jnp.float32)
            out_ref[pl.ds(HALF + c_rl * CH, CH), :] = ag_l[t].astype(
                jnp.float32
            )

        @functools.partial(
            pl.run_scoped, second_barrier=pltpu.SemaphoreType.REGULAR
        )
        def _(second_barrier):
            for nbr in (left, right):
                pl.semaphore_signal(
                    second_barrier, inc=1,
                    device_id=(nbr,), device_id_type=pl.DeviceIdType.MESH,
                )
            pl.semaphore_wait(second_barrier, 2)

    out2 = pl.pallas_call(
        body,
        out_shape=jax.ShapeDtypeStruct((SQ, D_MODEL), jnp.float32),
        in_specs=[pl.BlockSpec(memory_space=pltpu.VMEM)] * 5,
        out_specs=pl.BlockSpec(memory_space=pltpu.VMEM),
        scratch_shapes=[
            pltpu.VMEM((2, CH, D_MODEL), jnp.bfloat16),
            pltpu.VMEM((2, CH, D_MODEL), jnp.bfloat16),
            pltpu.VMEM((N_DEV - 1, CH, D_MODEL), jnp.bfloat16),
            pltpu.VMEM((N_DEV - 1, CH, D_MODEL), jnp.bfloat16),
            pltpu.VMEM((N_DEV - 1, CH, D_MODEL), jnp.bfloat16),
            pltpu.VMEM((N_DEV - 1, CH, D_MODEL), jnp.bfloat16),
            pltpu.SemaphoreType.DMA((2,)),
            pltpu.SemaphoreType.DMA((2,)),
            pltpu.SemaphoreType.DMA((N_DEV - 1,)),
            pltpu.SemaphoreType.DMA((N_DEV - 1,)),
            pltpu.SemaphoreType.DMA((N_DEV - 1, 2)),
            pltpu.SemaphoreType.DMA((N_DEV - 1, 2)),
        ],
        compiler_params=pltpu.CompilerParams(
            collective_id=0, vmem_limit_bytes=100 * 1024 * 1024
        ),
    )(x2, Wq_sl, K, V, Wo_sl)
    return out2[None]


# device time: 164755 ns/iter; 1.8578x vs baseline; 1.0684x over previous
import functools

import jax
import jax.numpy as jnp
from jax import lax
from jax.experimental import pallas as pl
from jax.experimental.pallas import tpu as pltpu

N_DEV = 16
SQ = 2048
D_MODEL = 1024
H_LOC = 8
DH = 128
WIN = 128
MB = 128
KBM = 384
HALF = SQ // 2
CH = HALF // N_DEV
NMB = HALF // MB
SCALE = 0.08838834764831843


def kernel(x, Wq, K_ext, V_ext, Wo):
    pos = lax.axis_index("i")
    x2 = x[0]
    K = K_ext[0]
    V = V_ext[0]
    Wq_sl = lax.dynamic_slice(Wq, (0, pos * D_MODEL), (D_MODEL, D_MODEL))
    Wo_sl = lax.dynamic_slice(Wo, (pos * D_MODEL, 0), (D_MODEL, D_MODEL))

    def body(x_ref, wq_ref, k_ref, v_ref, wo_ref, out_ref,
             stage_r, stage_l, rs_r, rs_l, ag_r, ag_l,
             send_sem_r, send_sem_l, rs_sems_r, rs_sems_l,
             ag_sems_r, ag_sems_l):
        my = lax.axis_index("i")
        left = jnp.mod(my - 1, N_DEV)
        right = jnp.mod(my + 1, N_DEV)

        barrier_sem = pltpu.get_barrier_semaphore()
        for nbr in (left, right):
            pl.semaphore_signal(
                barrier_sem, inc=1,
                device_id=(nbr,), device_id_type=pl.DeviceIdType.MESH,
            )

        def compute_mini(r0):
            ks = jnp.clip(r0 - WIN, 0, SQ - KBM)
            q_blk = jnp.dot(
                x_ref[pl.ds(r0, MB), :], wq_ref[...],
                preferred_element_type=jnp.float32,
            )
            heads = []
            rows = r0 + lax.broadcasted_iota(jnp.int32, (MB, KBM), 0)
            cols = ks + lax.broadcasted_iota(jnp.int32, (MB, KBM), 1)
            band = jnp.abs(rows - cols) <= WIN
            for h in range(H_LOC):
                q = q_blk[:, h * DH:(h + 1) * DH]
                k = k_ref[pl.ds(ks, KBM), h, :]
                s = lax.dot_general(
                    q, k, (((1,), (1,)), ((), ())),
                    preferred_element_type=jnp.float32,
                ) * SCALE
                s = jnp.where(band, s, -1e9)
                m = jnp.max(s, axis=1, keepdims=True)
                e = jnp.exp(s - m)
                w = e / jnp.sum(e, axis=1, keepdims=True)
                vv = v_ref[pl.ds(ks, KBM), h, :]
                heads.append(
                    jnp.dot(w, vv, preferred_element_type=jnp.float32)
                )
            ctx_blk = jnp.concatenate(heads, axis=1)
            out_ref[pl.ds(r0, MB), :] = jnp.dot(
                ctx_blk, wo_ref[...], preferred_element_type=jnp.float32
            )

        def mb_r(j):
            return jnp.mod(my // 2 - j, NMB) * MB

        def mb_l(j):
            return HALF + jnp.mod(my // 2 + j, NMB) * MB

        compute_mini(mb_r(0))
        compute_mini(mb_l(0))

        def mk(src, dst, ssem, rsem, nbr):
            return pltpu.make_async_remote_copy(
                src_ref=src, dst_ref=dst, send_sem=ssem, recv_sem=rsem,
                device_id=(nbr,), device_id_type=pl.DeviceIdType.MESH,
            )

        pl.semaphore_wait(barrier_sem, 2)

        stage_r[0, :, :] = out_ref[pl.ds(my * CH, CH), :].astype(jnp.bfloat16)
        stage_l[0, :, :] = out_ref[pl.ds(HALF + my * CH, CH), :].astype(
            jnp.bfloat16
        )
        prev_r = mk(stage_r.at[0], rs_r.at[0], send_sem_r.at[0],
                    rs_sems_r.at[0], right)
        prev_l = mk(stage_l.at[0], rs_l.at[0], send_sem_l.at[0],
                    rs_sems_l.at[0], left)
        prev_r.start()
        prev_l.start()
        for s in range(N_DEV - 1):
            if s == 0:
                compute_mini(mb_r(1))
                compute_mini(mb_l(1))
            if s % 2 == 1 and s <= 11:
                compute_mini(mb_r((s + 3) // 2))
            if s % 2 == 0 and 2 <= s <= 12:
                compute_mini(mb_l(s // 2 + 1))
            acc_cr = jnp.mod(my - s - 1, N_DEV)
            acc_cl = jnp.mod(my + s + 1, N_DEV)
            prev_r.wait_recv()
            val_r = (out_ref[pl.ds(acc_cr * CH, CH), :]
                     + rs_r[s].astype(jnp.float32))
            if s < N_DEV - 2:
                b = (s + 1) % 2
                stage_r[b, :, :] = val_r.astype(jnp.bfloat16)
                prev_r.wait_send()
                nxt_r = mk(stage_r.at[b], rs_r.at[s + 1], send_sem_r.at[0],
                           rs_sems_r.at[s + 1], right)
                nxt_r.start()
                prev_r = nxt_r
                prev_l.wait_recv()
                val_l = (out_ref[pl.ds(HALF + acc_cl * CH, CH), :]
                         + rs_l[s].astype(jnp.float32))
                stage_l[b, :, :] = val_l.astype(jnp.bfloat16)
                prev_l.wait_send()
                nxt_l = mk(stage_l.at[b], rs_l.at[s + 1], send_sem_l.at[0],
                           rs_sems_l.at[s + 1], left)
                nxt_l.start()
                prev_l = nxt_l
            else:
                prev_l.wait_recv()
                val_l = (out_ref[pl.ds(HALF + acc_cl * CH, CH), :]
                         + rs_l[s].astype(jnp.float32))
                out_ref[pl.ds(acc_cr * CH, CH), :] = val_r
                out_ref[pl.ds(HALF + acc_cl * CH, CH), :] = val_l
                stage_r[1, :, :] = val_r.astype(jnp.bfloat16)
                stage_l[1, :, :] = val_l.astype(jnp.bfloat16)
                prev_r.wait_send()
                prev_l.wait_send()

        CSPL = D_MODEL // 2
        cols = (slice(0, CSPL), slice(CSPL, D_MODEL))
        prevs = []
        for c in (0, 1):
            pr = mk(stage_r.at[1, :, cols[c]], ag_r.at[0, :, cols[c]],
                    send_sem_r.at[c], ag_sems_r.at[0, c], right)
            pl_ = mk(stage_l.at[1, :, cols[c]], ag_l.at[0, :, cols[c]],
                     send_sem_l.at[c], ag_sems_l.at[0, c], left)
            pr.start()
            pl_.start()
            prevs.append([pr, pl_])
        for t in range(N_DEV - 1):
            for c in (0, 1):
                pr, pl_ = prevs[c]
                pr.wait_recv()
                pr.wait_send()
                if t < N_DEV - 2:
                    nr = mk(ag_r.at[t, :, cols[c]],
                            ag_r.at[t + 1, :, cols[c]],
                            send_sem_r.at[c], ag_sems_r.at[t + 1, c],
                            right)
                    nr.start()
                pl_.wait_recv()
                pl_.wait_send()
                if t < N_DEV - 2:
                    nl = mk(ag_l.at[t, :, cols[c]],
                            ag_l.at[t + 1, :, cols[c]],
                            send_sem_l.at[c], ag_sems_l.at[t + 1, c],
                            left)
                    nl.start()
                    prevs[c] = [nr, nl]
            c_rr = jnp.mod(my - t, N_DEV)
            c_rl = jnp.mod(my + t, N_DEV)
            out_ref[pl.ds(c_rr * CH, CH), :] = ag_r[t].astype(jnp.float32)
            out_ref[pl.ds(HALF + c_rl * CH, CH), :] = ag_l[t].astype(
                jnp.float32
            )

        @functools.partial(
            pl.run_scoped, second_barrier=pltpu.SemaphoreType.REGULAR
        )
        def _(second_barrier):
            for nbr in (left, right):
                pl.semaphore_signal(
                    second_barrier, inc=1,
                    device_id=(nbr,), device_id_type=pl.DeviceIdType.MESH,
                )
            pl.semaphore_wait(second_barrier, 2)

    out2 = pl.pallas_call(
        body,
        out_shape=jax.ShapeDtypeStruct((SQ, D_MODEL), jnp.float32),
        in_specs=[pl.BlockSpec(memory_space=pltpu.VMEM)] * 5,
        out_specs=pl.BlockSpec(memory_space=pltpu.VMEM),
        scratch_shapes=[
            pltpu.VMEM((2, CH, D_MODEL), jnp.bfloat16),
            pltpu.VMEM((2, CH, D_MODEL), jnp.bfloat16),
            pltpu.VMEM((N_DEV - 1, CH, D_MODEL), jnp.bfloat16),
            pltpu.VMEM((N_DEV - 1, CH, D_MODEL), jnp.bfloat16),
            pltpu.VMEM((N_DEV - 1, CH, D_MODEL), jnp.bfloat16),
            pltpu.VMEM((N_DEV - 1, CH, D_MODEL), jnp.bfloat16),
            pltpu.SemaphoreType.DMA((2,)),
            pltpu.SemaphoreType.DMA((2,)),
            pltpu.SemaphoreType.DMA((N_DEV - 1,)),
            pltpu.SemaphoreType.DMA((N_DEV - 1,)),
            pltpu.SemaphoreType.DMA((N_DEV - 1, 2)),
            pltpu.SemaphoreType.DMA((N_DEV - 1, 2)),
        ],
        compiler_params=pltpu.CompilerParams(
            collective_id=0, vmem_limit_bytes=100 * 1024 * 1024
        ),
    )(x2, Wq_sl, K, V, Wo_sl)
    return out2[None]
